# baseline (speedup 1.0000x reference)
import jax
import jax.numpy as jnp
from jax.experimental import pallas as pl
from jax.experimental.pallas import tpu as pltpu

_N = 8 * 128 * 128 * 128
_COLS = 2048
_ROWS = _N // _COLS
_BLOCK_ROWS = 512
_GRID = _ROWS // _BLOCK_ROWS


def _masked_l1_kernel(yt_ref, yp_ref, m_ref, out_ref):
    i = pl.program_id(0)
    m = m_ref[...].astype(jnp.float32)
    s = jnp.sum(jnp.abs(yt_ref[...] - yp_ref[...]) * m)
    c = jnp.sum(m)

    @pl.when(i == 0)
    def _init():
        out_ref[0, 0] = s
        out_ref[0, 1] = c

    @pl.when(i != 0)
    def _acc():
        out_ref[0, 0] += s
        out_ref[0, 1] += c


def kernel(y_true, y_pred, lesion_mask):
    yt = y_true.reshape(_ROWS, _COLS)
    yp = y_pred.reshape(_ROWS, _COLS)
    m = lesion_mask.reshape(_ROWS, _COLS)

    in_spec = pl.BlockSpec((_BLOCK_ROWS, _COLS), lambda i: (i, 0))
    out = pl.pallas_call(
        _masked_l1_kernel,
        grid=(_GRID,),
        in_specs=[in_spec, in_spec, in_spec],
        out_specs=pl.BlockSpec(
            (1, 2), lambda i: (0, 0), memory_space=pltpu.SMEM
        ),
        out_shape=jax.ShapeDtypeStruct((1, 2), jnp.float32),
    )(yt, yp, m)
    return out[0, 0] / out[0, 1]


# speedup vs baseline: 1.0030x; 1.0030x over previous
import jax
import jax.numpy as jnp
from jax.experimental import pallas as pl
from jax.experimental.pallas import tpu as pltpu

_N = 8 * 128 * 128 * 128
_COLS = 2048
_ROWS = _N // _COLS
_BLOCK_ROWS = 512
_GRID = _ROWS // _BLOCK_ROWS
_SLAB = 8
_NSLAB = _BLOCK_ROWS // _SLAB


def _masked_l1_kernel(yt_ref, yp_ref, m_ref, out_ref):
    i = pl.program_id(0)

    def step(j, carry):
        s, c = carry
        yt = yt_ref[pl.ds(j * _SLAB, _SLAB), :]
        yp = yp_ref[pl.ds(j * _SLAB, _SLAB), :]
        m = m_ref[pl.ds(j * _SLAB, _SLAB), :].astype(jnp.float32)
        return s + jnp.abs(yt - yp) * m, c + m

    z = jnp.zeros((_SLAB, _COLS), jnp.float32)
    s, c = jax.lax.fori_loop(0, _NSLAB, step, (z, z))
    ps = jnp.sum(s)
    pc = jnp.sum(c)

    @pl.when(i == 0)
    def _init():
        out_ref[0, 0] = ps
        out_ref[0, 1] = pc

    @pl.when(i != 0)
    def _acc():
        out_ref[0, 0] += ps
        out_ref[0, 1] += pc


def kernel(y_true, y_pred, lesion_mask):
    yt = y_true.reshape(_ROWS, _COLS)
    yp = y_pred.reshape(_ROWS, _COLS)
    m = lesion_mask.reshape(_ROWS, _COLS)

    in_spec = pl.BlockSpec((_BLOCK_ROWS, _COLS), lambda i: (i, 0))
    out = pl.pallas_call(
        _masked_l1_kernel,
        grid=(_GRID,),
        in_specs=[in_spec, in_spec, in_spec],
        out_specs=pl.BlockSpec(
            (1, 2), lambda i: (0, 0), memory_space=pltpu.SMEM
        ),
        out_shape=jax.ShapeDtypeStruct((1, 2), jnp.float32),
    )(yt, yp, m)
    return out[0, 0] / out[0, 1]


# speedup vs baseline: 3.3003x; 3.2905x over previous
import jax
import jax.numpy as jnp
from jax.experimental import pallas as pl
from jax.experimental.pallas import tpu as pltpu

_N = 8 * 128 * 128 * 128
_COLS = 128
_ROWS = _N // _COLS
_BLOCK_ROWS = 8192
_GRID = _ROWS // _BLOCK_ROWS
_SLAB = 16
_NSLAB = _BLOCK_ROWS // _SLAB


def _masked_l1_kernel(yt_ref, yp_ref, m_ref, out_ref):
    i = pl.program_id(0)

    def step(j, carry):
        s, c = carry
        yt = yt_ref[pl.ds(j * _SLAB, _SLAB), :]
        yp = yp_ref[pl.ds(j * _SLAB, _SLAB), :]
        m = m_ref[pl.ds(j * _SLAB, _SLAB), :].astype(jnp.float32)
        return s + jnp.abs(yt - yp) * m, c + m

    z = jnp.zeros((_SLAB, _COLS), jnp.float32)
    s, c = jax.lax.fori_loop(0, _NSLAB, step, (z, z), unroll=4)
    ps = jnp.sum(s)
    pc = jnp.sum(c)

    @pl.when(i == 0)
    def _init():
        out_ref[0, 0] = ps
        out_ref[0, 1] = pc

    @pl.when(i != 0)
    def _acc():
        out_ref[0, 0] += ps
        out_ref[0, 1] += pc


def kernel(y_true, y_pred, lesion_mask):
    yt = y_true.reshape(_ROWS, _COLS)
    yp = y_pred.reshape(_ROWS, _COLS)
    m = lesion_mask.reshape(_ROWS, _COLS)

    in_spec = pl.BlockSpec((_BLOCK_ROWS, _COLS), lambda i: (i, 0))
    out = pl.pallas_call(
        _masked_l1_kernel,
        grid=(_GRID,),
        in_specs=[in_spec, in_spec, in_spec],
        out_specs=pl.BlockSpec(
            (1, 2), lambda i: (0, 0), memory_space=pltpu.SMEM
        ),
        out_shape=jax.ShapeDtypeStruct((1, 2), jnp.float32),
    )(yt, yp, m)
    return out[0, 0] / out[0, 1]
